# Initial kernel scaffold; baseline (speedup 1.0000x reference)
import jax
import jax.numpy as jnp
from jax.experimental import pallas as pl


def kernel(world_pos, prev_world_pos, node_type, mesh_pos, cells, mass, evals, evecs, gradX_rows, gradX_cols, gradX_vals, gradY_rows, gradY_cols, gradY_vals, L_rows, L_cols, L_vals, dn_first_w, dn_first_b, dn_t, dn_A_re, dn_A_im, dn_mlp_w1, dn_mlp_b1, dn_mlp_w2, dn_mlp_b2, dn_last_w, dn_last_b, enc_node_w1, enc_node_b1, enc_node_w2, enc_node_b2, enc_node_g, enc_node_be, enc_edge_w1, enc_edge_b1, enc_edge_w2, enc_edge_b2, enc_edge_g, enc_edge_be, mp_edge_w1, mp_edge_b1, mp_edge_w2, mp_edge_b2, mp_edge_g, mp_edge_be, mp_node_w1, mp_node_b1, mp_node_w2, mp_node_b2, mp_node_g, mp_node_be, dec_w1, dec_b1, dec_w2, dec_b2):
    raise NotImplementedError("write your pallas kernel here")


# speedup vs baseline: 1.4376x; 1.4376x over previous
import functools

import jax
import jax.numpy as jnp
from jax import lax
from jax.experimental import pallas as pl
from jax.experimental.pallas import tpu as pltpu
from jax.experimental.pallas import tpu_sc as plsc

N = 10000
E = 120000
EPAD = 122880
NNZ = 80000
NNZP = 81920
LAT = 128
NTYPES = 9
STEPS = 15
BLOCKS = 4
NC, NS, NW = 2, 16, 32
CHUNK = 128
ZROWS = 320
NACC = NW * ZROWS
TRASH = N
F32 = jnp.float32

def _sc_mesh():
    return plsc.VectorSubcoreMesh(core_axis_name="c", subcore_axis_name="s",
                                  num_cores=NC)


@functools.lru_cache(maxsize=None)
def _gather2(B, D):
    ch = B // (NW * CHUNK)

    @functools.partial(
        pl.kernel,
        out_type=(jax.ShapeDtypeStruct((B, D), F32),
                  jax.ShapeDtypeStruct((B, D), F32)),
        mesh=_sc_mesh(),
        scratch_types=[pltpu.VMEM((CHUNK,), jnp.int32),
                       pltpu.VMEM((CHUNK, D), F32),
                       pltpu.SemaphoreType.DMA],
    )
    def k(ta, ia, tb, ib, oa, ob, idx_v, rows_v, sem):
        wid = lax.axis_index("s") * NC + lax.axis_index("c")
        base = wid * (ch * CHUNK)

        def run(tbl, idx_hbm, out_hbm):
            def step(j, carry):
                off = base + j * CHUNK
                pltpu.sync_copy(idx_hbm.at[pl.ds(off, CHUNK)], idx_v)
                pltpu.async_copy(tbl.at[idx_v], rows_v, sem).wait()
                pltpu.sync_copy(rows_v, out_hbm.at[pl.ds(off, CHUNK)])
                return carry
            lax.fori_loop(0, ch, step, 0)

        run(ta, ia, oa)
        run(tb, ib, ob)

    return k


@functools.lru_cache(maxsize=None)
def _segsum(B):
    ch = B // (NW * CHUNK)

    @functools.partial(
        pl.kernel,
        out_type=jax.ShapeDtypeStruct((NC, NACC, LAT), F32),
        mesh=_sc_mesh(),
        scratch_types=[pltpu.VMEM((CHUNK,), jnp.int32),
                       pltpu.VMEM((CHUNK, LAT), F32),
                       pltpu.VMEM_SHARED((NACC, LAT), F32),
                       pltpu.SemaphoreType.DMA],
    )
    def k(src, ridx, zrow, out, idx_v, rows_v, acc, sem):
        cid = lax.axis_index("c")
        sid = lax.axis_index("s")
        wid = sid * NC + cid
        z0 = sid * (2 * ZROWS)
        pltpu.sync_copy(zrow, acc.at[pl.ds(z0, ZROWS)])
        pltpu.sync_copy(zrow, acc.at[pl.ds(z0 + ZROWS, ZROWS)])
        plsc.subcore_barrier()

        def step(j, carry):
            off = wid * (ch * CHUNK) + j * CHUNK
            pltpu.sync_copy(ridx.at[pl.ds(off, CHUNK)], idx_v)
            pltpu.sync_copy(src.at[pl.ds(off, CHUNK)], rows_v)
            pltpu.sync_copy(rows_v, acc.at[idx_v], add=True)
            return carry
        lax.fori_loop(0, ch, step, 0)
        plsc.subcore_barrier()
        pltpu.sync_copy(acc.at[pl.ds(z0, 2 * ZROWS)],
                        out.at[cid, pl.ds(z0, 2 * ZROWS)])

    return k


@functools.lru_cache(maxsize=None)
def _scatter_xy():
    ch = NNZP // (NS * CHUNK)

    @functools.partial(
        pl.kernel,
        out_type=jax.ShapeDtypeStruct((NC, NACC, LAT), F32),
        mesh=_sc_mesh(),
        scratch_types=[pltpu.VMEM((CHUNK,), jnp.int32),
                       pltpu.VMEM((CHUNK, LAT), F32),
                       pltpu.VMEM_SHARED((NACC, LAT), F32),
                       pltpu.SemaphoreType.DMA],
    )
    def k(srcx, rix, srcy, riy, zrow, out, idx_v, rows_v, acc, sem):
        cid = lax.axis_index("c")
        sid = lax.axis_index("s")
        z0 = sid * (2 * ZROWS)
        pltpu.sync_copy(zrow, acc.at[pl.ds(z0, ZROWS)])
        pltpu.sync_copy(zrow, acc.at[pl.ds(z0 + ZROWS, ZROWS)])
        plsc.subcore_barrier()

        def run(src, ridx):
            def step(j, carry):
                off = sid * (ch * CHUNK) + j * CHUNK
                pltpu.sync_copy(ridx.at[pl.ds(off, CHUNK)], idx_v)
                pltpu.sync_copy(src.at[pl.ds(off, CHUNK)], rows_v)
                pltpu.sync_copy(rows_v, acc.at[idx_v], add=True)
                return carry
            lax.fori_loop(0, ch, step, 0)

        @pl.when(cid == 0)
        def _():
            run(srcx, rix)

        @pl.when(cid == 1)
        def _():
            run(srcy, riy)

        plsc.subcore_barrier()
        pltpu.sync_copy(acc.at[pl.ds(z0, 2 * ZROWS)],
                        out.at[cid, pl.ds(z0, 2 * ZROWS)])

    return k


def _ln(h, g, be):
    mu = jnp.mean(h, axis=-1, keepdims=True)
    var = jnp.mean((h - mu) ** 2, axis=-1, keepdims=True)
    return (h - mu) / jnp.sqrt(var + 1e-5) * g + be


def _rowspec(blk, d):
    return pl.BlockSpec((blk, d), lambda i: (i, 0))


def _constspec(r, c):
    return pl.BlockSpec((r, c), lambda i: (0, 0))


def _nf_build(packed8, ntype8):
    blk = 2000

    def body(p_r, t_r, o_r):
        p = p_r[...]
        vel = p[:, 0:3] - p[:, 3:6]
        nt = t_r[...][:, 0:1]
        io = lax.broadcasted_iota(jnp.int32, (blk, 16), 1).astype(F32)
        oh = jnp.where((io >= 3.0) & (io < 12.0) & (io - 3.0 == nt), 1.0, 0.0)
        o_r[...] = jnp.concatenate([vel, jnp.zeros((blk, 13), F32)], 1) + oh

    return pl.pallas_call(
        body, grid=(N // blk,),
        in_specs=[_rowspec(blk, 8), _rowspec(blk, 8)],
        out_specs=_rowspec(blk, 16),
        out_shape=jax.ShapeDtypeStruct((N, 16), F32),
    )(packed8, ntype8)


def _stats(x, nvalid):
    rows, c = x.shape
    blk = 2048 if rows % 2048 == 0 else 2000

    def body(x_r, o_r):
        i = pl.program_id(0)

        @pl.when(i == 0)
        def _():
            o_r[...] = jnp.zeros_like(o_r)

        xv = x_r[...]
        rid = lax.broadcasted_iota(jnp.int32, (blk, 1), 0) + i * blk
        m = jnp.where(rid < nvalid, 1.0, 0.0).astype(F32)
        xm = xv * m
        o_r[0:1, 0:c] += jnp.sum(xm, axis=0, keepdims=True)
        o_r[1:2, 0:c] += jnp.sum(xm * xm, axis=0, keepdims=True)

    return pl.pallas_call(
        body, grid=(rows // blk,),
        in_specs=[_rowspec(blk, c)],
        out_specs=_constspec(8, 128),
        out_shape=jax.ShapeDtypeStruct((8, 128), F32),
        compiler_params=pltpu.CompilerParams(dimension_semantics=("arbitrary",)),
    )(x)


def _norm_from_stats(xv, st_r, c, nvalid):
    s = st_r[0:1, 0:c]
    s2 = st_r[1:2, 0:c]
    mean = s / nvalid
    std = jnp.maximum(jnp.sqrt(jnp.maximum(s2 / nvalid - mean * mean, 0.0)), 1e-8)
    return (xv - mean) / std


def _nf_apply(nf16, st, fw16, fb, nvalid):
    blk = 2000

    def body(x_r, st_r, w_r, b_r, on_r, ox_r):
        xn = _norm_from_stats(x_r[...], st_r[...], 16, nvalid)
        on_r[...] = xn
        ox_r[...] = xn @ w_r[...] + b_r[...]

    return pl.pallas_call(
        body, grid=(N // blk,),
        in_specs=[_rowspec(blk, 16), _constspec(8, 128),
                  _constspec(16, LAT), _constspec(1, LAT)],
        out_specs=(_rowspec(blk, 16), _rowspec(blk, LAT)),
        out_shape=(jax.ShapeDtypeStruct((N, 16), F32),
                   jax.ShapeDtypeStruct((N, LAT), F32)),
    )(nf16, st, fw16, fb)


def _ef_build(es, er):
    blk = 2048

    def body(s_r, r_r, o_r):
        rel = s_r[...][:, 0:8] - r_r[...][:, 0:8]
        rw = rel[:, 0:3]
        rm = rel[:, 3:5]
        nw = jnp.sqrt(jnp.sum(rw * rw, axis=-1, keepdims=True))
        nm = jnp.sqrt(jnp.sum(rm * rm, axis=-1, keepdims=True))
        o_r[...] = jnp.concatenate(
            [rw, nw, rm, nm, jnp.zeros((blk, 1), F32)], 1)

    return pl.pallas_call(
        body, grid=(EPAD // blk,),
        in_specs=[_rowspec(blk, LAT), _rowspec(blk, LAT)],
        out_specs=_rowspec(blk, 8),
        out_shape=jax.ShapeDtypeStruct((EPAD, 8), F32),
    )(es, er)


def _enc_edge(ef8, st, w1p, b1, w2, b2, g, be, nvalid):
    blk = 2048

    def body(x_r, st_r, w1_r, b1_r, w2_r, b2_r, g_r, be_r, o_r):
        xn = _norm_from_stats(x_r[...], st_r[...], 8, nvalid)
        h = jnp.maximum(xn @ w1_r[...] + b1_r[...], 0.0)
        o_r[...] = _ln(h @ w2_r[...] + b2_r[...], g_r[...], be_r[...])

    return pl.pallas_call(
        body, grid=(EPAD // blk,),
        in_specs=[_rowspec(blk, 8), _constspec(8, 128), _constspec(8, LAT),
                  _constspec(1, LAT), _constspec(LAT, LAT), _constspec(1, LAT),
                  _constspec(1, LAT), _constspec(1, LAT)],
        out_specs=_rowspec(blk, LAT),
        out_shape=jax.ShapeDtypeStruct((EPAD, LAT), F32),
    )(ef8, st, w1p, b1, w2, b2, g, be)


def _spec_mm(evecs, x, mass8):
    blk = 2000

    def body(e_r, x_r, m_r, o_r):
        i = pl.program_id(0)

        @pl.when(i == 0)
        def _():
            o_r[...] = jnp.zeros_like(o_r)

        xm = x_r[...] * m_r[...][:, 0:1]
        o_r[...] += lax.dot_general(e_r[...], xm, (((0,), (0,)), ((), ())),
                                    preferred_element_type=F32)

    return pl.pallas_call(
        body, grid=(N // blk,),
        in_specs=[_rowspec(blk, LAT), _rowspec(blk, LAT), _rowspec(blk, 8)],
        out_specs=_constspec(LAT, LAT),
        out_shape=jax.ShapeDtypeStruct((LAT, LAT), F32),
        compiler_params=pltpu.CompilerParams(dimension_semantics=("arbitrary",)),
    )(evecs, x, mass8)


def _xd_mm(evecs, spec, filt):
    blk = 2000

    def body(e_r, s_r, f_r, o_r):
        o_r[...] = e_r[...] @ (s_r[...] * f_r[...])

    return pl.pallas_call(
        body, grid=(N // blk,),
        in_specs=[_rowspec(blk, LAT), _constspec(LAT, LAT),
                  _constspec(LAT, LAT)],
        out_specs=_rowspec(blk, LAT),
        out_shape=jax.ShapeDtypeStruct((N, LAT), F32),
    )(evecs, spec, filt)


def _scale2(tx, vx, ty, vy):
    blk = 2048

    def body(tx_r, vx_r, ty_r, vy_r, ox_r, oy_r):
        ox_r[...] = tx_r[...] * vx_r[...]
        oy_r[...] = ty_r[...] * vy_r[...]

    return pl.pallas_call(
        body, grid=(NNZP // blk,),
        in_specs=[_rowspec(blk, LAT), _rowspec(blk, 1),
                  _rowspec(blk, LAT), _rowspec(blk, 1)],
        out_specs=(_rowspec(blk, LAT), _rowspec(blk, LAT)),
        out_shape=(jax.ShapeDtypeStruct((NNZP, LAT), F32),
                   jax.ShapeDtypeStruct((NNZP, LAT), F32)),
    )(tx, vx, ty, vy)


def _diff_block(x, xd, gx, gy, ar, ai, w1a, w1b, w1c, b1, w2, b2):
    blk = 2000

    def body(x_r, xd_r, gx_r, gy_r, ar_r, ai_r, w1a_r, w1b_r, w1c_r,
             b1_r, w2_r, b2_r, o_r):
        gxv = gx_r[...]
        gyv = gy_r[...]
        arv = ar_r[...]
        aiv = ai_r[...]
        bx = gxv @ arv - gyv @ aiv
        by = gxv @ aiv + gyv @ arv
        gf = jnp.tanh(gxv * bx + gyv * by)
        h = jnp.maximum(
            x_r[...] @ w1a_r[...] + xd_r[...] @ w1b_r[...]
            + gf @ w1c_r[...] + b1_r[...], 0.0)
        o_r[...] = x_r[...] + h @ w2_r[...] + b2_r[...]

    cs = _constspec(LAT, LAT)
    return pl.pallas_call(
        body, grid=(N // blk,),
        in_specs=[_rowspec(blk, LAT)] * 4 + [cs] * 5
                 + [_constspec(1, LAT), cs, _constspec(1, LAT)],
        out_specs=_rowspec(blk, LAT),
        out_shape=jax.ShapeDtypeStruct((N, LAT), F32),
    )(x, xd, gx, gy, ar, ai, w1a, w1b, w1c, b1, w2, b2)


def _node_enc(x, nf16n, lw16, lb16, w1a, w1b, b1, w2, b2, g, be, wbn, wcn):
    blk = 2000

    def body(x_r, nf_r, lw_r, lb_r, w1a_r, w1b_r, b1_r, w2_r, b2_r,
             g_r, be_r, wb_r, wc_r, oxv_r, op_r, oq_r):
        pred16 = x_r[...] @ lw_r[...] + lb_r[...]
        u = nf_r[...] @ w1a_r[...] + pred16 @ w1b_r[...] + b1_r[...]
        h = jnp.maximum(u, 0.0)
        xv = _ln(h @ w2_r[...] + b2_r[...], g_r[...], be_r[...])
        oxv_r[...] = xv
        op_r[...] = xv @ wb_r[...]
        oq_r[...] = xv @ wc_r[...]

    cs = _constspec(LAT, LAT)
    return pl.pallas_call(
        body, grid=(N // blk,),
        in_specs=[_rowspec(blk, LAT), _rowspec(blk, 16),
                  _constspec(LAT, 16), _constspec(1, 16),
                  _constspec(16, LAT), _constspec(16, LAT), _constspec(1, LAT),
                  cs, _constspec(1, LAT), _constspec(1, LAT),
                  _constspec(1, LAT), cs, cs],
        out_specs=(_rowspec(blk, LAT),) * 3,
        out_shape=(jax.ShapeDtypeStruct((N, LAT), F32),) * 3,
    )(x, nf16n, lw16, lb16, w1a, w1b, b1, w2, b2, g, be, wbn, wcn)


def _edge_step(ev, xsp, xrq, w1a, b1, w2, b2, g, be):
    blk = 2048

    def body(ev_r, xs_r, xr_r, w1_r, b1_r, w2_r, b2_r, g_r, be_r, o_r):
        h = jnp.maximum(
            ev_r[...] @ w1_r[...] + xs_r[...] + xr_r[...] + b1_r[...], 0.0)
        o_r[...] = ev_r[...] + _ln(h @ w2_r[...] + b2_r[...],
                                   g_r[...], be_r[...])

    cs = _constspec(LAT, LAT)
    return pl.pallas_call(
        body, grid=(EPAD // blk,),
        in_specs=[_rowspec(blk, LAT)] * 3
                 + [cs, _constspec(1, LAT), cs, _constspec(1, LAT),
                    _constspec(1, LAT), _constspec(1, LAT)],
        out_specs=_rowspec(blk, LAT),
        out_shape=jax.ShapeDtypeStruct((EPAD, LAT), F32),
    )(ev, xsp, xrq, w1a, b1, w2, b2, g, be)


def _node_step(xv, pt0, pt1, v1a, v1b, b1, w2, b2, g, be, wbn, wcn):
    blk = 2000

    def body(xv_r, p0_r, p1_r, v1a_r, v1b_r, b1_r, w2_r, b2_r, g_r, be_r,
             wb_r, wc_r, oxv_r, op_r, oq_r):
        agg = p0_r[...] + p1_r[...]
        h = jnp.maximum(
            xv_r[...] @ v1a_r[...] + agg @ v1b_r[...] + b1_r[...], 0.0)
        xvn = xv_r[...] + _ln(h @ w2_r[...] + b2_r[...], g_r[...], be_r[...])
        oxv_r[...] = xvn
        op_r[...] = xvn @ wb_r[...]
        oq_r[...] = xvn @ wc_r[...]

    cs = _constspec(LAT, LAT)
    return pl.pallas_call(
        body, grid=(N // blk,),
        in_specs=[_rowspec(blk, LAT)] * 3
                 + [cs, cs, _constspec(1, LAT), cs, _constspec(1, LAT),
                    _constspec(1, LAT), _constspec(1, LAT), cs, cs],
        out_specs=(_rowspec(blk, LAT),) * 3,
        out_shape=(jax.ShapeDtypeStruct((N, LAT), F32),) * 3,
    )(xv, pt0, pt1, v1a, v1b, b1, w2, b2, g, be, wbn, wcn)


def _decode(xv, w1, b1, w2p, b2p):
    blk = 2000

    def body(x_r, w1_r, b1_r, w2_r, b2_r, o_r):
        h = jnp.maximum(x_r[...] @ w1_r[...] + b1_r[...], 0.0)
        o_r[...] = h @ w2_r[...] + b2_r[...]

    return pl.pallas_call(
        body, grid=(N // blk,),
        in_specs=[_rowspec(blk, LAT), _constspec(LAT, LAT),
                  _constspec(1, LAT), _constspec(LAT, 8), _constspec(1, 8)],
        out_specs=_rowspec(blk, 8),
        out_shape=jax.ShapeDtypeStruct((N, 8), F32),
    )(xv, w1, b1, w2p, b2p)


def _row(v):
    return v.reshape(1, -1)


def _pad_idx(idx, size, fill):
    return jnp.concatenate(
        [idx.astype(jnp.int32),
         jnp.full((size - idx.shape[0],), fill, jnp.int32)])


def kernel(world_pos, prev_world_pos, node_type, mesh_pos, cells, mass,
           evals, evecs,
           gradX_rows, gradX_cols, gradX_vals,
           gradY_rows, gradY_cols, gradY_vals,
           L_rows, L_cols, L_vals,
           dn_first_w, dn_first_b, dn_t, dn_A_re, dn_A_im,
           dn_mlp_w1, dn_mlp_b1, dn_mlp_w2, dn_mlp_b2,
           dn_last_w, dn_last_b,
           enc_node_w1, enc_node_b1, enc_node_w2, enc_node_b2,
           enc_node_g, enc_node_be,
           enc_edge_w1, enc_edge_b1, enc_edge_w2, enc_edge_b2,
           enc_edge_g, enc_edge_be,
           mp_edge_w1, mp_edge_b1, mp_edge_w2, mp_edge_b2,
           mp_edge_g, mp_edge_be,
           mp_node_w1, mp_node_b1, mp_node_w2, mp_node_b2,
           mp_node_g, mp_node_be,
           dec_w1, dec_b1, dec_w2, dec_b2):
    wp0 = world_pos[0]
    pwp0 = prev_world_pos[0]
    mp0 = mesh_pos[0]
    c = cells[0].astype(jnp.int32)
    snd = jnp.concatenate([c[:, 0], c[:, 1], c[:, 2],
                           c[:, 1], c[:, 2], c[:, 0]])
    rcv = jnp.concatenate([c[:, 1], c[:, 2], c[:, 0],
                           c[:, 0], c[:, 1], c[:, 2]])
    snd_g = _pad_idx(snd, EPAD, 0)
    rcv_g = _pad_idx(rcv, EPAD, 0)
    rcv_s = _pad_idx(rcv, EPAD, TRASH)
    colx_g = _pad_idx(gradX_cols, NNZP, 0)
    coly_g = _pad_idx(gradY_cols, NNZP, 0)
    rowx_s = _pad_idx(gradX_rows, NNZP, TRASH)
    rowy_s = _pad_idx(gradY_rows, NNZP, TRASH)
    vcolx = jnp.pad(gradX_vals, (0, NNZP - NNZ))[:, None]
    vcoly = jnp.pad(gradY_vals, (0, NNZP - NNZ))[:, None]
    zrow = jnp.zeros((ZROWS, LAT), F32)

    packed8 = jnp.concatenate([wp0, pwp0, jnp.zeros((N, 2), F32)], 1)
    nt8 = jnp.broadcast_to(node_type[0].astype(F32), (N, 8))
    coords128 = jnp.concatenate([wp0, mp0, jnp.zeros((N, 123), F32)], 1)
    mass8 = jnp.broadcast_to(mass[0][:, None], (N, 8))

    fw16 = jnp.pad(dn_first_w, ((0, 4), (0, 0)))
    fb = _row(dn_first_b)
    lw16 = jnp.pad(dn_last_w, ((0, 0), (0, 4)))
    lb16 = _row(jnp.pad(dn_last_b, (0, 4)))
    enw1a = jnp.pad(enc_node_w1[:12], ((0, 4), (0, 0)))
    enw1b = jnp.pad(enc_node_w1[12:], ((0, 4), (0, 0)))
    eew1 = jnp.pad(enc_edge_w1, ((0, 1), (0, 0)))
    dw2p = jnp.pad(dec_w2, ((0, 0), (0, 5)))
    db2p = _row(jnp.pad(dec_b2, (0, 5)))

    nf16 = _nf_build(packed8, nt8)
    st_n = _stats(nf16, N)
    nf16n, x = _nf_apply(nf16, st_n, fw16, fb, N)

    es, er = _gather2(EPAD, LAT)(coords128, snd_g, coords128, rcv_g)
    ef8 = _ef_build(es, er)
    st_e = _stats(ef8, E)
    ev = _enc_edge(ef8, st_e, eew1, _row(enc_edge_b1), enc_edge_w2,
                   _row(enc_edge_b2), _row(enc_edge_g), _row(enc_edge_be), E)

    for b in range(BLOCKS):
        t = jnp.abs(dn_t[b]) + 1e-8
        filt = jnp.exp(-evals[0][:, None] * t[None, :])
        spec = _spec_mm(evecs[0], x, mass8)
        xd = _xd_mm(evecs[0], spec, filt)
        tx, ty = _gather2(NNZP, LAT)(xd, colx_g, xd, coly_g)
        txs, tys = _scale2(tx, vcolx, ty, vcoly)
        gxy = _scatter_xy()(txs, rowx_s, tys, rowy_s, zrow)
        x = _diff_block(x, xd, gxy[0, :N], gxy[1, :N],
                        dn_A_re[b], dn_A_im[b],
                        dn_mlp_w1[b][:LAT], dn_mlp_w1[b][LAT:2 * LAT],
                        dn_mlp_w1[b][2 * LAT:], _row(dn_mlp_b1[b]),
                        dn_mlp_w2[b], _row(dn_mlp_b2[b]))

    xv, p, q = _node_enc(
        x, nf16n, lw16, lb16, enw1a, enw1b, _row(enc_node_b1),
        enc_node_w2, _row(enc_node_b2), _row(enc_node_g), _row(enc_node_be),
        mp_edge_w1[0][LAT:2 * LAT], mp_edge_w1[0][2 * LAT:])

    for s in range(STEPS):
        xsp, xrq = _gather2(EPAD, LAT)(p, snd_g, q, rcv_g)
        ev = _edge_step(ev, xsp, xrq, mp_edge_w1[s][:LAT],
                        _row(mp_edge_b1[s]), mp_edge_w2[s],
                        _row(mp_edge_b2[s]), _row(mp_edge_g[s]),
                        _row(mp_edge_be[s]))
        parts = _segsum(EPAD)(ev, rcv_s, zrow)
        nxt = min(s + 1, STEPS - 1)
        xv, p, q = _node_step(
            xv, parts[0, :N], parts[1, :N],
            mp_node_w1[s][:LAT], mp_node_w1[s][LAT:], _row(mp_node_b1[s]),
            mp_node_w2[s], _row(mp_node_b2[s]), _row(mp_node_g[s]),
            _row(mp_node_be[s]),
            mp_edge_w1[nxt][LAT:2 * LAT], mp_edge_w1[nxt][2 * LAT:])

    out8 = _decode(xv, dec_w1, _row(dec_b1), dw2p, db2p)
    return out8[:, :3][None]


# speedup vs baseline: 1.5652x; 1.0888x over previous
import functools

import jax
import jax.numpy as jnp
from jax import lax
from jax.experimental import pallas as pl
from jax.experimental.pallas import tpu as pltpu
from jax.experimental.pallas import tpu_sc as plsc

N = 10000
E = 120000
EPAD = 122880
NNZ = 80000
NNZP = 81920
LAT = 128
NTYPES = 9
STEPS = 15
BLOCKS = 4
NC, NS, NW = 2, 16, 32
CHUNK = 128
ZROWS = 320
NACC = NW * ZROWS
TRASH = N
F32 = jnp.float32

def _sc_mesh():
    return plsc.VectorSubcoreMesh(core_axis_name="c", subcore_axis_name="s",
                                  num_cores=NC)


KB = 5


@functools.lru_cache(maxsize=None)
def _gather2(B, D):
    ch = B // (NW * CHUNK)

    @functools.partial(
        pl.kernel,
        out_type=(jax.ShapeDtypeStruct((B, D), F32),
                  jax.ShapeDtypeStruct((B, D), F32)),
        mesh=_sc_mesh(),
        scratch_types=[pltpu.VMEM((ch * CHUNK,), jnp.int32)]
                      + [pltpu.VMEM((CHUNK, D), F32)] * KB
                      + [pltpu.SemaphoreType.DMA, pltpu.SemaphoreType.DMA],
    )
    def k(ta, ia, tb, ib, oa, ob, idx_all, *rest):
        bufs = rest[:KB]
        gsem, wsem = rest[KB], rest[KB + 1]
        wid = lax.axis_index("s") * NC + lax.axis_index("c")
        base = wid * (ch * CHUNK)

        def run(tbl, idx_hbm, out_hbm):
            pltpu.sync_copy(idx_hbm.at[pl.ds(base, ch * CHUNK)], idx_all)

            def group(g, carry):
                gd, wd = [], []
                for b in range(KB):
                    j = g * KB + b
                    src = tbl.at[idx_all.at[pl.ds(j * CHUNK, CHUNK)]]
                    gd.append(pltpu.async_copy(src, bufs[b], gsem))
                for d in gd:
                    d.wait()
                for b in range(KB):
                    j = g * KB + b
                    dst = out_hbm.at[pl.ds(base + j * CHUNK, CHUNK)]
                    wd.append(pltpu.async_copy(bufs[b], dst, wsem))
                for d in wd:
                    d.wait()
                return carry
            lax.fori_loop(0, ch // KB, group, 0)

        run(ta, ia, oa)
        run(tb, ib, ob)

    return k


@functools.lru_cache(maxsize=None)
def _segsum(B):
    ch = B // (NW * CHUNK)

    @functools.partial(
        pl.kernel,
        out_type=jax.ShapeDtypeStruct((NC, NACC, LAT), F32),
        mesh=_sc_mesh(),
        scratch_types=[pltpu.VMEM((CHUNK,), jnp.int32),
                       pltpu.VMEM((CHUNK, LAT), F32),
                       pltpu.VMEM_SHARED((NACC, LAT), F32),
                       pltpu.SemaphoreType.DMA],
    )
    def k(src, ridx, zrow, out, idx_v, rows_v, acc, sem):
        cid = lax.axis_index("c")
        sid = lax.axis_index("s")
        wid = sid * NC + cid
        z0 = sid * (2 * ZROWS)
        pltpu.sync_copy(zrow, acc.at[pl.ds(z0, ZROWS)])
        pltpu.sync_copy(zrow, acc.at[pl.ds(z0 + ZROWS, ZROWS)])
        plsc.subcore_barrier()

        def step(j, carry):
            off = wid * (ch * CHUNK) + j * CHUNK
            pltpu.sync_copy(ridx.at[pl.ds(off, CHUNK)], idx_v)
            pltpu.sync_copy(src.at[pl.ds(off, CHUNK)], rows_v)
            pltpu.sync_copy(rows_v, acc.at[idx_v], add=True)
            return carry
        lax.fori_loop(0, ch, step, 0)
        plsc.subcore_barrier()
        pltpu.sync_copy(acc.at[pl.ds(z0, 2 * ZROWS)],
                        out.at[cid, pl.ds(z0, 2 * ZROWS)])

    return k


@functools.lru_cache(maxsize=None)
def _scatter_xy():
    ch = NNZP // (NS * CHUNK)

    @functools.partial(
        pl.kernel,
        out_type=jax.ShapeDtypeStruct((NC, NACC, LAT), F32),
        mesh=_sc_mesh(),
        scratch_types=[pltpu.VMEM((CHUNK,), jnp.int32),
                       pltpu.VMEM((CHUNK, LAT), F32),
                       pltpu.VMEM_SHARED((NACC, LAT), F32),
                       pltpu.SemaphoreType.DMA],
    )
    def k(srcx, rix, srcy, riy, zrow, out, idx_v, rows_v, acc, sem):
        cid = lax.axis_index("c")
        sid = lax.axis_index("s")
        z0 = sid * (2 * ZROWS)
        pltpu.sync_copy(zrow, acc.at[pl.ds(z0, ZROWS)])
        pltpu.sync_copy(zrow, acc.at[pl.ds(z0 + ZROWS, ZROWS)])
        plsc.subcore_barrier()

        def run(src, ridx):
            def step(j, carry):
                off = sid * (ch * CHUNK) + j * CHUNK
                pltpu.sync_copy(ridx.at[pl.ds(off, CHUNK)], idx_v)
                pltpu.sync_copy(src.at[pl.ds(off, CHUNK)], rows_v)
                pltpu.sync_copy(rows_v, acc.at[idx_v], add=True)
                return carry
            lax.fori_loop(0, ch, step, 0)

        @pl.when(cid == 0)
        def _():
            run(srcx, rix)

        @pl.when(cid == 1)
        def _():
            run(srcy, riy)

        plsc.subcore_barrier()
        pltpu.sync_copy(acc.at[pl.ds(z0, 2 * ZROWS)],
                        out.at[cid, pl.ds(z0, 2 * ZROWS)])

    return k


def _ln(h, g, be):
    mu = jnp.mean(h, axis=-1, keepdims=True)
    var = jnp.mean((h - mu) ** 2, axis=-1, keepdims=True)
    return (h - mu) / jnp.sqrt(var + 1e-5) * g + be


def _rowspec(blk, d):
    return pl.BlockSpec((blk, d), lambda i: (i, 0))


def _constspec(r, c):
    return pl.BlockSpec((r, c), lambda i: (0, 0))


def _nf_build(packed8, ntype8):
    blk = 2000

    def body(p_r, t_r, o_r):
        p = p_r[...]
        vel = p[:, 0:3] - p[:, 3:6]
        nt = t_r[...][:, 0:1]
        io = lax.broadcasted_iota(jnp.int32, (blk, 16), 1).astype(F32)
        oh = jnp.where((io >= 3.0) & (io < 12.0) & (io - 3.0 == nt), 1.0, 0.0)
        o_r[...] = jnp.concatenate([vel, jnp.zeros((blk, 13), F32)], 1) + oh

    return pl.pallas_call(
        body, grid=(N // blk,),
        in_specs=[_rowspec(blk, 8), _rowspec(blk, 8)],
        out_specs=_rowspec(blk, 16),
        out_shape=jax.ShapeDtypeStruct((N, 16), F32),
    )(packed8, ntype8)


def _stats(x, nvalid):
    rows, c = x.shape
    blk = 2048 if rows % 2048 == 0 else 2000

    def body(x_r, o_r):
        i = pl.program_id(0)

        @pl.when(i == 0)
        def _():
            o_r[...] = jnp.zeros_like(o_r)

        xv = x_r[...]
        rid = lax.broadcasted_iota(jnp.int32, (blk, 1), 0) + i * blk
        m = jnp.where(rid < nvalid, 1.0, 0.0).astype(F32)
        xm = xv * m
        o_r[0:1, 0:c] += jnp.sum(xm, axis=0, keepdims=True)
        o_r[1:2, 0:c] += jnp.sum(xm * xm, axis=0, keepdims=True)

    return pl.pallas_call(
        body, grid=(rows // blk,),
        in_specs=[_rowspec(blk, c)],
        out_specs=_constspec(8, 128),
        out_shape=jax.ShapeDtypeStruct((8, 128), F32),
        compiler_params=pltpu.CompilerParams(dimension_semantics=("arbitrary",)),
    )(x)


def _norm_from_stats(xv, st_r, c, nvalid):
    s = st_r[0:1, 0:c]
    s2 = st_r[1:2, 0:c]
    mean = s / nvalid
    std = jnp.maximum(jnp.sqrt(jnp.maximum(s2 / nvalid - mean * mean, 0.0)), 1e-8)
    return (xv - mean) / std


def _nf_apply(nf16, st, fw16, fb, nvalid):
    blk = 2000

    def body(x_r, st_r, w_r, b_r, on_r, ox_r):
        xn = _norm_from_stats(x_r[...], st_r[...], 16, nvalid)
        on_r[...] = xn
        ox_r[...] = xn @ w_r[...] + b_r[...]

    return pl.pallas_call(
        body, grid=(N // blk,),
        in_specs=[_rowspec(blk, 16), _constspec(8, 128),
                  _constspec(16, LAT), _constspec(1, LAT)],
        out_specs=(_rowspec(blk, 16), _rowspec(blk, LAT)),
        out_shape=(jax.ShapeDtypeStruct((N, 16), F32),
                   jax.ShapeDtypeStruct((N, LAT), F32)),
    )(nf16, st, fw16, fb)


def _ef_build(es, er):
    blk = 2048

    def body(s_r, r_r, o_r):
        rel = s_r[...][:, 0:8] - r_r[...][:, 0:8]
        rw = rel[:, 0:3]
        rm = rel[:, 3:5]
        nw = jnp.sqrt(jnp.sum(rw * rw, axis=-1, keepdims=True))
        nm = jnp.sqrt(jnp.sum(rm * rm, axis=-1, keepdims=True))
        o_r[...] = jnp.concatenate(
            [rw, nw, rm, nm, jnp.zeros((blk, 1), F32)], 1)

    return pl.pallas_call(
        body, grid=(EPAD // blk,),
        in_specs=[_rowspec(blk, LAT), _rowspec(blk, LAT)],
        out_specs=_rowspec(blk, 8),
        out_shape=jax.ShapeDtypeStruct((EPAD, 8), F32),
    )(es, er)


def _enc_edge(ef8, st, w1p, b1, w2, b2, g, be, nvalid):
    blk = 2048

    def body(x_r, st_r, w1_r, b1_r, w2_r, b2_r, g_r, be_r, o_r):
        xn = _norm_from_stats(x_r[...], st_r[...], 8, nvalid)
        h = jnp.maximum(xn @ w1_r[...] + b1_r[...], 0.0)
        o_r[...] = _ln(h @ w2_r[...] + b2_r[...], g_r[...], be_r[...])

    return pl.pallas_call(
        body, grid=(EPAD // blk,),
        in_specs=[_rowspec(blk, 8), _constspec(8, 128), _constspec(8, LAT),
                  _constspec(1, LAT), _constspec(LAT, LAT), _constspec(1, LAT),
                  _constspec(1, LAT), _constspec(1, LAT)],
        out_specs=_rowspec(blk, LAT),
        out_shape=jax.ShapeDtypeStruct((EPAD, LAT), F32),
    )(ef8, st, w1p, b1, w2, b2, g, be)


def _spec_mm(evecs, x, mass8):
    blk = 2000

    def body(e_r, x_r, m_r, o_r):
        i = pl.program_id(0)

        @pl.when(i == 0)
        def _():
            o_r[...] = jnp.zeros_like(o_r)

        xm = x_r[...] * m_r[...][:, 0:1]
        o_r[...] += lax.dot_general(e_r[...], xm, (((0,), (0,)), ((), ())),
                                    preferred_element_type=F32)

    return pl.pallas_call(
        body, grid=(N // blk,),
        in_specs=[_rowspec(blk, LAT), _rowspec(blk, LAT), _rowspec(blk, 8)],
        out_specs=_constspec(LAT, LAT),
        out_shape=jax.ShapeDtypeStruct((LAT, LAT), F32),
        compiler_params=pltpu.CompilerParams(dimension_semantics=("arbitrary",)),
    )(evecs, x, mass8)


def _xd_mm(evecs, spec, filt):
    blk = 2000

    def body(e_r, s_r, f_r, o_r):
        o_r[...] = e_r[...] @ (s_r[...] * f_r[...])

    return pl.pallas_call(
        body, grid=(N // blk,),
        in_specs=[_rowspec(blk, LAT), _constspec(LAT, LAT),
                  _constspec(LAT, LAT)],
        out_specs=_rowspec(blk, LAT),
        out_shape=jax.ShapeDtypeStruct((N, LAT), F32),
    )(evecs, spec, filt)


def _scale2(tx, vx, ty, vy):
    blk = 2048

    def body(tx_r, vx_r, ty_r, vy_r, ox_r, oy_r):
        ox_r[...] = tx_r[...] * vx_r[...]
        oy_r[...] = ty_r[...] * vy_r[...]

    return pl.pallas_call(
        body, grid=(NNZP // blk,),
        in_specs=[_rowspec(blk, LAT), _rowspec(blk, 1),
                  _rowspec(blk, LAT), _rowspec(blk, 1)],
        out_specs=(_rowspec(blk, LAT), _rowspec(blk, LAT)),
        out_shape=(jax.ShapeDtypeStruct((NNZP, LAT), F32),
                   jax.ShapeDtypeStruct((NNZP, LAT), F32)),
    )(tx, vx, ty, vy)


def _diff_block(x, xd, gx, gy, ar, ai, w1a, w1b, w1c, b1, w2, b2):
    blk = 2000

    def body(x_r, xd_r, gx_r, gy_r, ar_r, ai_r, w1a_r, w1b_r, w1c_r,
             b1_r, w2_r, b2_r, o_r):
        gxv = gx_r[...]
        gyv = gy_r[...]
        arv = ar_r[...]
        aiv = ai_r[...]
        bx = gxv @ arv - gyv @ aiv
        by = gxv @ aiv + gyv @ arv
        gf = jnp.tanh(gxv * bx + gyv * by)
        h = jnp.maximum(
            x_r[...] @ w1a_r[...] + xd_r[...] @ w1b_r[...]
            + gf @ w1c_r[...] + b1_r[...], 0.0)
        o_r[...] = x_r[...] + h @ w2_r[...] + b2_r[...]

    cs = _constspec(LAT, LAT)
    return pl.pallas_call(
        body, grid=(N // blk,),
        in_specs=[_rowspec(blk, LAT)] * 4 + [cs] * 5
                 + [_constspec(1, LAT), cs, _constspec(1, LAT)],
        out_specs=_rowspec(blk, LAT),
        out_shape=jax.ShapeDtypeStruct((N, LAT), F32),
    )(x, xd, gx, gy, ar, ai, w1a, w1b, w1c, b1, w2, b2)


def _node_enc(x, nf16n, lw16, lb16, w1a, w1b, b1, w2, b2, g, be, wbn, wcn):
    blk = 2000

    def body(x_r, nf_r, lw_r, lb_r, w1a_r, w1b_r, b1_r, w2_r, b2_r,
             g_r, be_r, wb_r, wc_r, oxv_r, op_r, oq_r):
        pred16 = x_r[...] @ lw_r[...] + lb_r[...]
        u = nf_r[...] @ w1a_r[...] + pred16 @ w1b_r[...] + b1_r[...]
        h = jnp.maximum(u, 0.0)
        xv = _ln(h @ w2_r[...] + b2_r[...], g_r[...], be_r[...])
        oxv_r[...] = xv
        op_r[...] = xv @ wb_r[...]
        oq_r[...] = xv @ wc_r[...]

    cs = _constspec(LAT, LAT)
    return pl.pallas_call(
        body, grid=(N // blk,),
        in_specs=[_rowspec(blk, LAT), _rowspec(blk, 16),
                  _constspec(LAT, 16), _constspec(1, 16),
                  _constspec(16, LAT), _constspec(16, LAT), _constspec(1, LAT),
                  cs, _constspec(1, LAT), _constspec(1, LAT),
                  _constspec(1, LAT), cs, cs],
        out_specs=(_rowspec(blk, LAT),) * 3,
        out_shape=(jax.ShapeDtypeStruct((N, LAT), F32),) * 3,
    )(x, nf16n, lw16, lb16, w1a, w1b, b1, w2, b2, g, be, wbn, wcn)


def _edge_step(ev, xsp, xrq, w1a, b1, w2, b2, g, be):
    blk = 2048

    def body(ev_r, xs_r, xr_r, w1_r, b1_r, w2_r, b2_r, g_r, be_r, o_r):
        h = jnp.maximum(
            ev_r[...] @ w1_r[...] + xs_r[...] + xr_r[...] + b1_r[...], 0.0)
        o_r[...] = ev_r[...] + _ln(h @ w2_r[...] + b2_r[...],
                                   g_r[...], be_r[...])

    cs = _constspec(LAT, LAT)
    return pl.pallas_call(
        body, grid=(EPAD // blk,),
        in_specs=[_rowspec(blk, LAT)] * 3
                 + [cs, _constspec(1, LAT), cs, _constspec(1, LAT),
                    _constspec(1, LAT), _constspec(1, LAT)],
        out_specs=_rowspec(blk, LAT),
        out_shape=jax.ShapeDtypeStruct((EPAD, LAT), F32),
    )(ev, xsp, xrq, w1a, b1, w2, b2, g, be)


def _node_step(xv, pt0, pt1, v1a, v1b, b1, w2, b2, g, be, wbn, wcn):
    blk = 2000

    def body(xv_r, p0_r, p1_r, v1a_r, v1b_r, b1_r, w2_r, b2_r, g_r, be_r,
             wb_r, wc_r, oxv_r, op_r, oq_r):
        agg = p0_r[...] + p1_r[...]
        h = jnp.maximum(
            xv_r[...] @ v1a_r[...] + agg @ v1b_r[...] + b1_r[...], 0.0)
        xvn = xv_r[...] + _ln(h @ w2_r[...] + b2_r[...], g_r[...], be_r[...])
        oxv_r[...] = xvn
        op_r[...] = xvn @ wb_r[...]
        oq_r[...] = xvn @ wc_r[...]

    cs = _constspec(LAT, LAT)
    return pl.pallas_call(
        body, grid=(N // blk,),
        in_specs=[_rowspec(blk, LAT)] * 3
                 + [cs, cs, _constspec(1, LAT), cs, _constspec(1, LAT),
                    _constspec(1, LAT), _constspec(1, LAT), cs, cs],
        out_specs=(_rowspec(blk, LAT),) * 3,
        out_shape=(jax.ShapeDtypeStruct((N, LAT), F32),) * 3,
    )(xv, pt0, pt1, v1a, v1b, b1, w2, b2, g, be, wbn, wcn)


def _decode(xv, w1, b1, w2p, b2p):
    blk = 2000

    def body(x_r, w1_r, b1_r, w2_r, b2_r, o_r):
        h = jnp.maximum(x_r[...] @ w1_r[...] + b1_r[...], 0.0)
        o_r[...] = h @ w2_r[...] + b2_r[...]

    return pl.pallas_call(
        body, grid=(N // blk,),
        in_specs=[_rowspec(blk, LAT), _constspec(LAT, LAT),
                  _constspec(1, LAT), _constspec(LAT, 8), _constspec(1, 8)],
        out_specs=_rowspec(blk, 8),
        out_shape=jax.ShapeDtypeStruct((N, 8), F32),
    )(xv, w1, b1, w2p, b2p)


def _row(v):
    return v.reshape(1, -1)


def _pad_idx(idx, size, fill):
    return jnp.concatenate(
        [idx.astype(jnp.int32),
         jnp.full((size - idx.shape[0],), fill, jnp.int32)])


def kernel(world_pos, prev_world_pos, node_type, mesh_pos, cells, mass,
           evals, evecs,
           gradX_rows, gradX_cols, gradX_vals,
           gradY_rows, gradY_cols, gradY_vals,
           L_rows, L_cols, L_vals,
           dn_first_w, dn_first_b, dn_t, dn_A_re, dn_A_im,
           dn_mlp_w1, dn_mlp_b1, dn_mlp_w2, dn_mlp_b2,
           dn_last_w, dn_last_b,
           enc_node_w1, enc_node_b1, enc_node_w2, enc_node_b2,
           enc_node_g, enc_node_be,
           enc_edge_w1, enc_edge_b1, enc_edge_w2, enc_edge_b2,
           enc_edge_g, enc_edge_be,
           mp_edge_w1, mp_edge_b1, mp_edge_w2, mp_edge_b2,
           mp_edge_g, mp_edge_be,
           mp_node_w1, mp_node_b1, mp_node_w2, mp_node_b2,
           mp_node_g, mp_node_be,
           dec_w1, dec_b1, dec_w2, dec_b2):
    wp0 = world_pos[0]
    pwp0 = prev_world_pos[0]
    mp0 = mesh_pos[0]
    c = cells[0].astype(jnp.int32)
    snd = jnp.concatenate([c[:, 0], c[:, 1], c[:, 2],
                           c[:, 1], c[:, 2], c[:, 0]])
    rcv = jnp.concatenate([c[:, 1], c[:, 2], c[:, 0],
                           c[:, 0], c[:, 1], c[:, 2]])
    snd_g = _pad_idx(snd, EPAD, 0)
    rcv_g = _pad_idx(rcv, EPAD, 0)
    rcv_s = _pad_idx(rcv, EPAD, TRASH)
    colx_g = _pad_idx(gradX_cols, NNZP, 0)
    coly_g = _pad_idx(gradY_cols, NNZP, 0)
    rowx_s = _pad_idx(gradX_rows, NNZP, TRASH)
    rowy_s = _pad_idx(gradY_rows, NNZP, TRASH)
    vcolx = jnp.pad(gradX_vals, (0, NNZP - NNZ))[:, None]
    vcoly = jnp.pad(gradY_vals, (0, NNZP - NNZ))[:, None]
    zrow = jnp.zeros((ZROWS, LAT), F32)

    packed8 = jnp.concatenate([wp0, pwp0, jnp.zeros((N, 2), F32)], 1)
    nt8 = jnp.broadcast_to(node_type[0].astype(F32), (N, 8))
    coords128 = jnp.concatenate([wp0, mp0, jnp.zeros((N, 123), F32)], 1)
    mass8 = jnp.broadcast_to(mass[0][:, None], (N, 8))

    fw16 = jnp.pad(dn_first_w, ((0, 4), (0, 0)))
    fb = _row(dn_first_b)
    lw16 = jnp.pad(dn_last_w, ((0, 0), (0, 4)))
    lb16 = _row(jnp.pad(dn_last_b, (0, 4)))
    enw1a = jnp.pad(enc_node_w1[:12], ((0, 4), (0, 0)))
    enw1b = jnp.pad(enc_node_w1[12:], ((0, 4), (0, 0)))
    eew1 = jnp.pad(enc_edge_w1, ((0, 1), (0, 0)))
    dw2p = jnp.pad(dec_w2, ((0, 0), (0, 5)))
    db2p = _row(jnp.pad(dec_b2, (0, 5)))

    nf16 = _nf_build(packed8, nt8)
    st_n = _stats(nf16, N)
    nf16n, x = _nf_apply(nf16, st_n, fw16, fb, N)

    es, er = _gather2(EPAD, LAT)(coords128, snd_g, coords128, rcv_g)
    ef8 = _ef_build(es, er)
    st_e = _stats(ef8, E)
    ev = _enc_edge(ef8, st_e, eew1, _row(enc_edge_b1), enc_edge_w2,
                   _row(enc_edge_b2), _row(enc_edge_g), _row(enc_edge_be), E)

    for b in range(BLOCKS):
        t = jnp.abs(dn_t[b]) + 1e-8
        filt = jnp.exp(-evals[0][:, None] * t[None, :])
        spec = _spec_mm(evecs[0], x, mass8)
        xd = _xd_mm(evecs[0], spec, filt)
        tx, ty = _gather2(NNZP, LAT)(xd, colx_g, xd, coly_g)
        txs, tys = _scale2(tx, vcolx, ty, vcoly)
        gxy = _scatter_xy()(txs, rowx_s, tys, rowy_s, zrow)
        x = _diff_block(x, xd, gxy[0, :N], gxy[1, :N],
                        dn_A_re[b], dn_A_im[b],
                        dn_mlp_w1[b][:LAT], dn_mlp_w1[b][LAT:2 * LAT],
                        dn_mlp_w1[b][2 * LAT:], _row(dn_mlp_b1[b]),
                        dn_mlp_w2[b], _row(dn_mlp_b2[b]))

    xv, p, q = _node_enc(
        x, nf16n, lw16, lb16, enw1a, enw1b, _row(enc_node_b1),
        enc_node_w2, _row(enc_node_b2), _row(enc_node_g), _row(enc_node_be),
        mp_edge_w1[0][LAT:2 * LAT], mp_edge_w1[0][2 * LAT:])

    for s in range(STEPS):
        xsp, xrq = _gather2(EPAD, LAT)(p, snd_g, q, rcv_g)
        ev = _edge_step(ev, xsp, xrq, mp_edge_w1[s][:LAT],
                        _row(mp_edge_b1[s]), mp_edge_w2[s],
                        _row(mp_edge_b2[s]), _row(mp_edge_g[s]),
                        _row(mp_edge_be[s]))
        parts = _segsum(EPAD)(ev, rcv_s, zrow)
        nxt = min(s + 1, STEPS - 1)
        xv, p, q = _node_step(
            xv, parts[0, :N], parts[1, :N],
            mp_node_w1[s][:LAT], mp_node_w1[s][LAT:], _row(mp_node_b1[s]),
            mp_node_w2[s], _row(mp_node_b2[s]), _row(mp_node_g[s]),
            _row(mp_node_be[s]),
            mp_edge_w1[nxt][LAT:2 * LAT], mp_edge_w1[nxt][2 * LAT:])

    out8 = _decode(xv, dec_w1, _row(dec_b1), dw2p, db2p)
    return out8[:, :3][None]


# speedup vs baseline: 1.5773x; 1.0078x over previous
import functools

import jax
import jax.numpy as jnp
from jax import lax
from jax.experimental import pallas as pl
from jax.experimental.pallas import tpu as pltpu
from jax.experimental.pallas import tpu_sc as plsc

N = 10000
E = 120000
EPAD = 122880
NNZ = 80000
NNZP = 81920
LAT = 128
NTYPES = 9
STEPS = 15
BLOCKS = 4
NC, NS, NW = 2, 16, 32
CHUNK = 128
ZROWS = 320
NACC = NW * ZROWS
TRASH = N
F32 = jnp.float32

def _sc_mesh():
    return plsc.VectorSubcoreMesh(core_axis_name="c", subcore_axis_name="s",
                                  num_cores=NC)


@functools.lru_cache(maxsize=None)
def _gather2(B, D):
    ch = B // (NW * CHUNK)
    kb = next(k for k in (3, 2, 1) if ch % (2 * k) == 0)
    npair = ch // (2 * kb)

    @functools.partial(
        pl.kernel,
        out_type=(jax.ShapeDtypeStruct((B, D), F32),
                  jax.ShapeDtypeStruct((B, D), F32)),
        mesh=_sc_mesh(),
        scratch_types=[pltpu.VMEM((ch * CHUNK,), jnp.int32)]
                      + [pltpu.VMEM((CHUNK, D), F32)] * (2 * kb)
                      + [pltpu.SemaphoreType.DMA] * 4,
    )
    def k(ta, ia, tb, ib, oa, ob, idx_all, *rest):
        bufa, bufb = rest[:kb], rest[kb:2 * kb]
        gsa, gsb, wsa, wsb = rest[2 * kb:2 * kb + 4]
        wid = lax.axis_index("s") * NC + lax.axis_index("c")
        base = wid * (ch * CHUNK)

        def run(tbl, idx_hbm, out_hbm):
            pltpu.sync_copy(idx_hbm.at[pl.ds(base, ch * CHUNK)], idx_all)

            def fire_g(g, bufs, sem):
                for b in range(kb):
                    j = g * kb + b
                    src = tbl.at[idx_all.at[pl.ds(j * CHUNK, CHUNK)]]
                    pltpu.async_copy(src, bufs[b], sem)

            def drain_g(bufs, sem):
                for b in range(kb):
                    pltpu.make_async_copy(
                        out_hbm.at[pl.ds(base, CHUNK)], bufs[b], sem).wait()

            def fire_w(g, bufs, sem):
                for b in range(kb):
                    j = g * kb + b
                    dst = out_hbm.at[pl.ds(base + j * CHUNK, CHUNK)]
                    pltpu.async_copy(bufs[b], dst, sem)

            def drain_w(bufs, sem):
                for b in range(kb):
                    pltpu.make_async_copy(
                        bufs[b], out_hbm.at[pl.ds(base, CHUNK)], sem).wait()

            fire_g(0, bufa, gsa)

            def pair(p, carry):
                ga = 2 * p
                gn = jnp.minimum(2 * p + 2, 2 * npair - 1)
                drain_g(bufa, gsa)
                fire_w(ga, bufa, wsa)
                fire_g(ga + 1, bufb, gsb)
                drain_g(bufb, gsb)
                fire_w(ga + 1, bufb, wsb)
                drain_w(bufa, wsa)
                fire_g(gn, bufa, gsa)
                drain_w(bufb, wsb)
                return carry
            lax.fori_loop(0, npair, pair, 0)
            drain_g(bufa, gsa)

        run(ta, ia, oa)
        run(tb, ib, ob)

    return k


@functools.lru_cache(maxsize=None)
def _segsum(B):
    ch = B // (NW * CHUNK)

    @functools.partial(
        pl.kernel,
        out_type=jax.ShapeDtypeStruct((NC, NACC, LAT), F32),
        mesh=_sc_mesh(),
        scratch_types=[pltpu.VMEM((CHUNK,), jnp.int32),
                       pltpu.VMEM((CHUNK,), jnp.int32),
                       pltpu.VMEM((CHUNK, LAT), F32),
                       pltpu.VMEM((CHUNK, LAT), F32),
                       pltpu.VMEM_SHARED((NACC, LAT), F32),
                       pltpu.SemaphoreType.DMA, pltpu.SemaphoreType.DMA],
    )
    def k(src, ridx, zrow, out, idx0, idx1, rows0, rows1, acc, sma, smb):
        cid = lax.axis_index("c")
        sid = lax.axis_index("s")
        wid = sid * NC + cid
        z0 = sid * (2 * ZROWS)
        base = wid * (ch * CHUNK)
        pltpu.sync_copy(zrow, acc.at[pl.ds(z0, ZROWS)])
        pltpu.sync_copy(zrow, acc.at[pl.ds(z0 + ZROWS, ZROWS)])
        plsc.subcore_barrier()

        def fire(c, iv, rv, sem):
            off = base + c * CHUNK
            pltpu.async_copy(ridx.at[pl.ds(off, CHUNK)], iv, sem)
            pltpu.async_copy(src.at[pl.ds(off, CHUNK)], rv, sem)

        def drain(iv, rv, sem):
            pltpu.make_async_copy(ridx.at[pl.ds(base, CHUNK)], iv, sem).wait()
            pltpu.make_async_copy(src.at[pl.ds(base, CHUNK)], rv, sem).wait()

        fire(0, idx0, rows0, sma)

        def step(h, carry):
            cn = jnp.minimum(2 * h + 2, ch - 1)
            drain(idx0, rows0, sma)
            fire(2 * h + 1, idx1, rows1, smb)
            pltpu.sync_copy(rows0, acc.at[idx0], add=True)
            drain(idx1, rows1, smb)
            fire(cn, idx0, rows0, sma)
            pltpu.sync_copy(rows1, acc.at[idx1], add=True)
            return carry
        lax.fori_loop(0, ch // 2, step, 0)
        drain(idx0, rows0, sma)
        plsc.subcore_barrier()
        pltpu.sync_copy(acc.at[pl.ds(z0, 2 * ZROWS)],
                        out.at[cid, pl.ds(z0, 2 * ZROWS)])

    return k


@functools.lru_cache(maxsize=None)
def _scatter_xy():
    ch = NNZP // (NS * CHUNK)

    @functools.partial(
        pl.kernel,
        out_type=jax.ShapeDtypeStruct((NC, NACC, LAT), F32),
        mesh=_sc_mesh(),
        scratch_types=[pltpu.VMEM((CHUNK,), jnp.int32),
                       pltpu.VMEM((CHUNK,), jnp.int32),
                       pltpu.VMEM((CHUNK, LAT), F32),
                       pltpu.VMEM((CHUNK, LAT), F32),
                       pltpu.VMEM_SHARED((NACC, LAT), F32),
                       pltpu.SemaphoreType.DMA, pltpu.SemaphoreType.DMA],
    )
    def k(srcx, rix, srcy, riy, zrow, out, idx0, idx1, rows0, rows1, acc,
          sma, smb):
        cid = lax.axis_index("c")
        sid = lax.axis_index("s")
        z0 = sid * (2 * ZROWS)
        base = sid * (ch * CHUNK)
        pltpu.sync_copy(zrow, acc.at[pl.ds(z0, ZROWS)])
        pltpu.sync_copy(zrow, acc.at[pl.ds(z0 + ZROWS, ZROWS)])
        plsc.subcore_barrier()

        def run(src, ridx):
            def fire(c, iv, rv, sem):
                off = base + c * CHUNK
                pltpu.async_copy(ridx.at[pl.ds(off, CHUNK)], iv, sem)
                pltpu.async_copy(src.at[pl.ds(off, CHUNK)], rv, sem)

            def drain(iv, rv, sem):
                pltpu.make_async_copy(ridx.at[pl.ds(base, CHUNK)], iv,
                                      sem).wait()
                pltpu.make_async_copy(src.at[pl.ds(base, CHUNK)], rv,
                                      sem).wait()

            fire(0, idx0, rows0, sma)

            def step(h, carry):
                cn = jnp.minimum(2 * h + 2, ch - 1)
                drain(idx0, rows0, sma)
                fire(2 * h + 1, idx1, rows1, smb)
                pltpu.sync_copy(rows0, acc.at[idx0], add=True)
                drain(idx1, rows1, smb)
                fire(cn, idx0, rows0, sma)
                pltpu.sync_copy(rows1, acc.at[idx1], add=True)
                return carry
            lax.fori_loop(0, ch // 2, step, 0)
            drain(idx0, rows0, sma)

        @pl.when(cid == 0)
        def _():
            run(srcx, rix)

        @pl.when(cid == 1)
        def _():
            run(srcy, riy)

        plsc.subcore_barrier()
        pltpu.sync_copy(acc.at[pl.ds(z0, 2 * ZROWS)],
                        out.at[cid, pl.ds(z0, 2 * ZROWS)])

    return k


def _ln(h, g, be):
    mu = jnp.mean(h, axis=-1, keepdims=True)
    var = jnp.mean((h - mu) ** 2, axis=-1, keepdims=True)
    return (h - mu) / jnp.sqrt(var + 1e-5) * g + be


def _rowspec(blk, d):
    return pl.BlockSpec((blk, d), lambda i: (i, 0))


def _constspec(r, c):
    return pl.BlockSpec((r, c), lambda i: (0, 0))


def _nf_build(packed8, ntype8):
    blk = 2000

    def body(p_r, t_r, o_r):
        p = p_r[...]
        vel = p[:, 0:3] - p[:, 3:6]
        nt = t_r[...][:, 0:1]
        io = lax.broadcasted_iota(jnp.int32, (blk, 16), 1).astype(F32)
        oh = jnp.where((io >= 3.0) & (io < 12.0) & (io - 3.0 == nt), 1.0, 0.0)
        o_r[...] = jnp.concatenate([vel, jnp.zeros((blk, 13), F32)], 1) + oh

    return pl.pallas_call(
        body, grid=(N // blk,),
        in_specs=[_rowspec(blk, 8), _rowspec(blk, 8)],
        out_specs=_rowspec(blk, 16),
        out_shape=jax.ShapeDtypeStruct((N, 16), F32),
    )(packed8, ntype8)


def _stats(x, nvalid):
    rows, c = x.shape
    blk = 2048 if rows % 2048 == 0 else 2000

    def body(x_r, o_r):
        i = pl.program_id(0)

        @pl.when(i == 0)
        def _():
            o_r[...] = jnp.zeros_like(o_r)

        xv = x_r[...]
        rid = lax.broadcasted_iota(jnp.int32, (blk, 1), 0) + i * blk
        m = jnp.where(rid < nvalid, 1.0, 0.0).astype(F32)
        xm = xv * m
        o_r[0:1, 0:c] += jnp.sum(xm, axis=0, keepdims=True)
        o_r[1:2, 0:c] += jnp.sum(xm * xm, axis=0, keepdims=True)

    return pl.pallas_call(
        body, grid=(rows // blk,),
        in_specs=[_rowspec(blk, c)],
        out_specs=_constspec(8, 128),
        out_shape=jax.ShapeDtypeStruct((8, 128), F32),
        compiler_params=pltpu.CompilerParams(dimension_semantics=("arbitrary",)),
    )(x)


def _norm_from_stats(xv, st_r, c, nvalid):
    s = st_r[0:1, 0:c]
    s2 = st_r[1:2, 0:c]
    mean = s / nvalid
    std = jnp.maximum(jnp.sqrt(jnp.maximum(s2 / nvalid - mean * mean, 0.0)), 1e-8)
    return (xv - mean) / std


def _nf_apply(nf16, st, fw16, fb, nvalid):
    blk = 2000

    def body(x_r, st_r, w_r, b_r, on_r, ox_r):
        xn = _norm_from_stats(x_r[...], st_r[...], 16, nvalid)
        on_r[...] = xn
        ox_r[...] = xn @ w_r[...] + b_r[...]

    return pl.pallas_call(
        body, grid=(N // blk,),
        in_specs=[_rowspec(blk, 16), _constspec(8, 128),
                  _constspec(16, LAT), _constspec(1, LAT)],
        out_specs=(_rowspec(blk, 16), _rowspec(blk, LAT)),
        out_shape=(jax.ShapeDtypeStruct((N, 16), F32),
                   jax.ShapeDtypeStruct((N, LAT), F32)),
    )(nf16, st, fw16, fb)


def _ef_build(es, er):
    blk = 2048

    def body(s_r, r_r, o_r):
        rel = s_r[...][:, 0:8] - r_r[...][:, 0:8]
        rw = rel[:, 0:3]
        rm = rel[:, 3:5]
        nw = jnp.sqrt(jnp.sum(rw * rw, axis=-1, keepdims=True))
        nm = jnp.sqrt(jnp.sum(rm * rm, axis=-1, keepdims=True))
        o_r[...] = jnp.concatenate(
            [rw, nw, rm, nm, jnp.zeros((blk, 1), F32)], 1)

    return pl.pallas_call(
        body, grid=(EPAD // blk,),
        in_specs=[_rowspec(blk, LAT), _rowspec(blk, LAT)],
        out_specs=_rowspec(blk, 8),
        out_shape=jax.ShapeDtypeStruct((EPAD, 8), F32),
    )(es, er)


def _enc_edge(ef8, st, w1p, b1, w2, b2, g, be, nvalid):
    blk = 2048

    def body(x_r, st_r, w1_r, b1_r, w2_r, b2_r, g_r, be_r, o_r):
        xn = _norm_from_stats(x_r[...], st_r[...], 8, nvalid)
        h = jnp.maximum(xn @ w1_r[...] + b1_r[...], 0.0)
        o_r[...] = _ln(h @ w2_r[...] + b2_r[...], g_r[...], be_r[...])

    return pl.pallas_call(
        body, grid=(EPAD // blk,),
        in_specs=[_rowspec(blk, 8), _constspec(8, 128), _constspec(8, LAT),
                  _constspec(1, LAT), _constspec(LAT, LAT), _constspec(1, LAT),
                  _constspec(1, LAT), _constspec(1, LAT)],
        out_specs=_rowspec(blk, LAT),
        out_shape=jax.ShapeDtypeStruct((EPAD, LAT), F32),
    )(ef8, st, w1p, b1, w2, b2, g, be)


def _spec_mm(evecs, x, mass8):
    blk = 2000

    def body(e_r, x_r, m_r, o_r):
        i = pl.program_id(0)

        @pl.when(i == 0)
        def _():
            o_r[...] = jnp.zeros_like(o_r)

        xm = x_r[...] * m_r[...][:, 0:1]
        o_r[...] += lax.dot_general(e_r[...], xm, (((0,), (0,)), ((), ())),
                                    preferred_element_type=F32)

    return pl.pallas_call(
        body, grid=(N // blk,),
        in_specs=[_rowspec(blk, LAT), _rowspec(blk, LAT), _rowspec(blk, 8)],
        out_specs=_constspec(LAT, LAT),
        out_shape=jax.ShapeDtypeStruct((LAT, LAT), F32),
        compiler_params=pltpu.CompilerParams(dimension_semantics=("arbitrary",)),
    )(evecs, x, mass8)


def _xd_mm(evecs, spec, filt):
    blk = 2000

    def body(e_r, s_r, f_r, o_r):
        o_r[...] = e_r[...] @ (s_r[...] * f_r[...])

    return pl.pallas_call(
        body, grid=(N // blk,),
        in_specs=[_rowspec(blk, LAT), _constspec(LAT, LAT),
                  _constspec(LAT, LAT)],
        out_specs=_rowspec(blk, LAT),
        out_shape=jax.ShapeDtypeStruct((N, LAT), F32),
    )(evecs, spec, filt)


def _scale2(tx, vx, ty, vy):
    blk = 2048

    def body(tx_r, vx_r, ty_r, vy_r, ox_r, oy_r):
        ox_r[...] = tx_r[...] * vx_r[...]
        oy_r[...] = ty_r[...] * vy_r[...]

    return pl.pallas_call(
        body, grid=(NNZP // blk,),
        in_specs=[_rowspec(blk, LAT), _rowspec(blk, 1),
                  _rowspec(blk, LAT), _rowspec(blk, 1)],
        out_specs=(_rowspec(blk, LAT), _rowspec(blk, LAT)),
        out_shape=(jax.ShapeDtypeStruct((NNZP, LAT), F32),
                   jax.ShapeDtypeStruct((NNZP, LAT), F32)),
    )(tx, vx, ty, vy)


def _diff_block(x, xd, gx, gy, ar, ai, w1a, w1b, w1c, b1, w2, b2):
    blk = 2000

    def body(x_r, xd_r, gx_r, gy_r, ar_r, ai_r, w1a_r, w1b_r, w1c_r,
             b1_r, w2_r, b2_r, o_r):
        gxv = gx_r[...]
        gyv = gy_r[...]
        arv = ar_r[...]
        aiv = ai_r[...]
        bx = gxv @ arv - gyv @ aiv
        by = gxv @ aiv + gyv @ arv
        gf = jnp.tanh(gxv * bx + gyv * by)
        h = jnp.maximum(
            x_r[...] @ w1a_r[...] + xd_r[...] @ w1b_r[...]
            + gf @ w1c_r[...] + b1_r[...], 0.0)
        o_r[...] = x_r[...] + h @ w2_r[...] + b2_r[...]

    cs = _constspec(LAT, LAT)
    return pl.pallas_call(
        body, grid=(N // blk,),
        in_specs=[_rowspec(blk, LAT)] * 4 + [cs] * 5
                 + [_constspec(1, LAT), cs, _constspec(1, LAT)],
        out_specs=_rowspec(blk, LAT),
        out_shape=jax.ShapeDtypeStruct((N, LAT), F32),
    )(x, xd, gx, gy, ar, ai, w1a, w1b, w1c, b1, w2, b2)


def _node_enc(x, nf16n, lw16, lb16, w1a, w1b, b1, w2, b2, g, be, wbn, wcn):
    blk = 2000

    def body(x_r, nf_r, lw_r, lb_r, w1a_r, w1b_r, b1_r, w2_r, b2_r,
             g_r, be_r, wb_r, wc_r, oxv_r, op_r, oq_r):
        pred16 = x_r[...] @ lw_r[...] + lb_r[...]
        u = nf_r[...] @ w1a_r[...] + pred16 @ w1b_r[...] + b1_r[...]
        h = jnp.maximum(u, 0.0)
        xv = _ln(h @ w2_r[...] + b2_r[...], g_r[...], be_r[...])
        oxv_r[...] = xv
        op_r[...] = xv @ wb_r[...]
        oq_r[...] = xv @ wc_r[...]

    cs = _constspec(LAT, LAT)
    return pl.pallas_call(
        body, grid=(N // blk,),
        in_specs=[_rowspec(blk, LAT), _rowspec(blk, 16),
                  _constspec(LAT, 16), _constspec(1, 16),
                  _constspec(16, LAT), _constspec(16, LAT), _constspec(1, LAT),
                  cs, _constspec(1, LAT), _constspec(1, LAT),
                  _constspec(1, LAT), cs, cs],
        out_specs=(_rowspec(blk, LAT),) * 3,
        out_shape=(jax.ShapeDtypeStruct((N, LAT), F32),) * 3,
    )(x, nf16n, lw16, lb16, w1a, w1b, b1, w2, b2, g, be, wbn, wcn)


def _edge_step(ev, xsp, xrq, w1a, b1, w2, b2, g, be):
    blk = 2048

    def body(ev_r, xs_r, xr_r, w1_r, b1_r, w2_r, b2_r, g_r, be_r, o_r):
        h = jnp.maximum(
            ev_r[...] @ w1_r[...] + xs_r[...] + xr_r[...] + b1_r[...], 0.0)
        o_r[...] = ev_r[...] + _ln(h @ w2_r[...] + b2_r[...],
                                   g_r[...], be_r[...])

    cs = _constspec(LAT, LAT)
    return pl.pallas_call(
        body, grid=(EPAD // blk,),
        in_specs=[_rowspec(blk, LAT)] * 3
                 + [cs, _constspec(1, LAT), cs, _constspec(1, LAT),
                    _constspec(1, LAT), _constspec(1, LAT)],
        out_specs=_rowspec(blk, LAT),
        out_shape=jax.ShapeDtypeStruct((EPAD, LAT), F32),
    )(ev, xsp, xrq, w1a, b1, w2, b2, g, be)


def _node_step(xv, pt0, pt1, v1a, v1b, b1, w2, b2, g, be, wbn, wcn):
    blk = 2000

    def body(xv_r, p0_r, p1_r, v1a_r, v1b_r, b1_r, w2_r, b2_r, g_r, be_r,
             wb_r, wc_r, oxv_r, op_r, oq_r):
        agg = p0_r[...] + p1_r[...]
        h = jnp.maximum(
            xv_r[...] @ v1a_r[...] + agg @ v1b_r[...] + b1_r[...], 0.0)
        xvn = xv_r[...] + _ln(h @ w2_r[...] + b2_r[...], g_r[...], be_r[...])
        oxv_r[...] = xvn
        op_r[...] = xvn @ wb_r[...]
        oq_r[...] = xvn @ wc_r[...]

    cs = _constspec(LAT, LAT)
    return pl.pallas_call(
        body, grid=(N // blk,),
        in_specs=[_rowspec(blk, LAT)] * 3
                 + [cs, cs, _constspec(1, LAT), cs, _constspec(1, LAT),
                    _constspec(1, LAT), _constspec(1, LAT), cs, cs],
        out_specs=(_rowspec(blk, LAT),) * 3,
        out_shape=(jax.ShapeDtypeStruct((N, LAT), F32),) * 3,
    )(xv, pt0, pt1, v1a, v1b, b1, w2, b2, g, be, wbn, wcn)


def _decode(xv, w1, b1, w2p, b2p):
    blk = 2000

    def body(x_r, w1_r, b1_r, w2_r, b2_r, o_r):
        h = jnp.maximum(x_r[...] @ w1_r[...] + b1_r[...], 0.0)
        o_r[...] = h @ w2_r[...] + b2_r[...]

    return pl.pallas_call(
        body, grid=(N // blk,),
        in_specs=[_rowspec(blk, LAT), _constspec(LAT, LAT),
                  _constspec(1, LAT), _constspec(LAT, 8), _constspec(1, 8)],
        out_specs=_rowspec(blk, 8),
        out_shape=jax.ShapeDtypeStruct((N, 8), F32),
    )(xv, w1, b1, w2p, b2p)


def _row(v):
    return v.reshape(1, -1)


def _pad_idx(idx, size, fill):
    return jnp.concatenate(
        [idx.astype(jnp.int32),
         jnp.full((size - idx.shape[0],), fill, jnp.int32)])


def kernel(world_pos, prev_world_pos, node_type, mesh_pos, cells, mass,
           evals, evecs,
           gradX_rows, gradX_cols, gradX_vals,
           gradY_rows, gradY_cols, gradY_vals,
           L_rows, L_cols, L_vals,
           dn_first_w, dn_first_b, dn_t, dn_A_re, dn_A_im,
           dn_mlp_w1, dn_mlp_b1, dn_mlp_w2, dn_mlp_b2,
           dn_last_w, dn_last_b,
           enc_node_w1, enc_node_b1, enc_node_w2, enc_node_b2,
           enc_node_g, enc_node_be,
           enc_edge_w1, enc_edge_b1, enc_edge_w2, enc_edge_b2,
           enc_edge_g, enc_edge_be,
           mp_edge_w1, mp_edge_b1, mp_edge_w2, mp_edge_b2,
           mp_edge_g, mp_edge_be,
           mp_node_w1, mp_node_b1, mp_node_w2, mp_node_b2,
           mp_node_g, mp_node_be,
           dec_w1, dec_b1, dec_w2, dec_b2):
    wp0 = world_pos[0]
    pwp0 = prev_world_pos[0]
    mp0 = mesh_pos[0]
    c = cells[0].astype(jnp.int32)
    snd = jnp.concatenate([c[:, 0], c[:, 1], c[:, 2],
                           c[:, 1], c[:, 2], c[:, 0]])
    rcv = jnp.concatenate([c[:, 1], c[:, 2], c[:, 0],
                           c[:, 0], c[:, 1], c[:, 2]])
    snd_g = _pad_idx(snd, EPAD, 0)
    rcv_g = _pad_idx(rcv, EPAD, 0)
    rcv_s = _pad_idx(rcv, EPAD, TRASH)
    colx_g = _pad_idx(gradX_cols, NNZP, 0)
    coly_g = _pad_idx(gradY_cols, NNZP, 0)
    rowx_s = _pad_idx(gradX_rows, NNZP, TRASH)
    rowy_s = _pad_idx(gradY_rows, NNZP, TRASH)
    vcolx = jnp.pad(gradX_vals, (0, NNZP - NNZ))[:, None]
    vcoly = jnp.pad(gradY_vals, (0, NNZP - NNZ))[:, None]
    zrow = jnp.zeros((ZROWS, LAT), F32)

    packed8 = jnp.concatenate([wp0, pwp0, jnp.zeros((N, 2), F32)], 1)
    nt8 = jnp.broadcast_to(node_type[0].astype(F32), (N, 8))
    coords128 = jnp.concatenate([wp0, mp0, jnp.zeros((N, 123), F32)], 1)
    mass8 = jnp.broadcast_to(mass[0][:, None], (N, 8))

    fw16 = jnp.pad(dn_first_w, ((0, 4), (0, 0)))
    fb = _row(dn_first_b)
    lw16 = jnp.pad(dn_last_w, ((0, 0), (0, 4)))
    lb16 = _row(jnp.pad(dn_last_b, (0, 4)))
    enw1a = jnp.pad(enc_node_w1[:12], ((0, 4), (0, 0)))
    enw1b = jnp.pad(enc_node_w1[12:], ((0, 4), (0, 0)))
    eew1 = jnp.pad(enc_edge_w1, ((0, 1), (0, 0)))
    dw2p = jnp.pad(dec_w2, ((0, 0), (0, 5)))
    db2p = _row(jnp.pad(dec_b2, (0, 5)))

    nf16 = _nf_build(packed8, nt8)
    st_n = _stats(nf16, N)
    nf16n, x = _nf_apply(nf16, st_n, fw16, fb, N)

    es, er = _gather2(EPAD, LAT)(coords128, snd_g, coords128, rcv_g)
    ef8 = _ef_build(es, er)
    st_e = _stats(ef8, E)
    ev = _enc_edge(ef8, st_e, eew1, _row(enc_edge_b1), enc_edge_w2,
                   _row(enc_edge_b2), _row(enc_edge_g), _row(enc_edge_be), E)

    for b in range(BLOCKS):
        t = jnp.abs(dn_t[b]) + 1e-8
        filt = jnp.exp(-evals[0][:, None] * t[None, :])
        spec = _spec_mm(evecs[0], x, mass8)
        xd = _xd_mm(evecs[0], spec, filt)
        tx, ty = _gather2(NNZP, LAT)(xd, colx_g, xd, coly_g)
        txs, tys = _scale2(tx, vcolx, ty, vcoly)
        gxy = _scatter_xy()(txs, rowx_s, tys, rowy_s, zrow)
        x = _diff_block(x, xd, gxy[0, :N], gxy[1, :N],
                        dn_A_re[b], dn_A_im[b],
                        dn_mlp_w1[b][:LAT], dn_mlp_w1[b][LAT:2 * LAT],
                        dn_mlp_w1[b][2 * LAT:], _row(dn_mlp_b1[b]),
                        dn_mlp_w2[b], _row(dn_mlp_b2[b]))

    xv, p, q = _node_enc(
        x, nf16n, lw16, lb16, enw1a, enw1b, _row(enc_node_b1),
        enc_node_w2, _row(enc_node_b2), _row(enc_node_g), _row(enc_node_be),
        mp_edge_w1[0][LAT:2 * LAT], mp_edge_w1[0][2 * LAT:])

    for s in range(STEPS):
        xsp, xrq = _gather2(EPAD, LAT)(p, snd_g, q, rcv_g)
        ev = _edge_step(ev, xsp, xrq, mp_edge_w1[s][:LAT],
                        _row(mp_edge_b1[s]), mp_edge_w2[s],
                        _row(mp_edge_b2[s]), _row(mp_edge_g[s]),
                        _row(mp_edge_be[s]))
        parts = _segsum(EPAD)(ev, rcv_s, zrow)
        nxt = min(s + 1, STEPS - 1)
        xv, p, q = _node_step(
            xv, parts[0, :N], parts[1, :N],
            mp_node_w1[s][:LAT], mp_node_w1[s][LAT:], _row(mp_node_b1[s]),
            mp_node_w2[s], _row(mp_node_b2[s]), _row(mp_node_g[s]),
            _row(mp_node_be[s]),
            mp_edge_w1[nxt][LAT:2 * LAT], mp_edge_w1[nxt][2 * LAT:])

    out8 = _decode(xv, dec_w1, _row(dec_b1), dw2p, db2p)
    return out8[:, :3][None]
